# baseline (speedup 1.0000x reference)
import jax
import jax.numpy as jnp
from jax.experimental import pallas as pl
from jax.experimental.pallas import tpu as pltpu

_N = 2708
_INS = 512
_OUTS = 64
_NM = 4
_LEAK = 0.2
_NP = 2816
_BN = 176
_HA = 128


def _cat_kernel(x_ref, adj_ref, W_ref, a1_ref, a2_ref, Wc_ref, bc_ref,
                out_ref, h_scr, r1_scr, r2_scr, c1_scr, c2_scr, cond_scr):
    i = pl.program_id(0)

    @pl.when(i == 0)
    def _prologue():
        x = x_ref[...]
        cond_scr[...] = (
            jnp.dot(x, Wc_ref[...], preferred_element_type=jnp.float32)
            + bc_ref[...]
        )
        xb = x.astype(jnp.bfloat16)
        rows = jax.lax.broadcasted_iota(jnp.int32, (_NP, 1), 0)
        cols = jax.lax.broadcasted_iota(jnp.int32, (1, _NP), 1)
        for m in range(_NM):
            h = jnp.dot(xb, W_ref[m].astype(jnp.bfloat16),
                        preferred_element_type=jnp.float32)
            h = jnp.where(rows < _N, h, 0.0)
            h_scr[m, :, :_OUTS] = h.astype(jnp.bfloat16)
            h_scr[m, :, _OUTS:] = jnp.ones((_NP, _HA - _OUTS), jnp.bfloat16)
            f1 = jnp.dot(h, a1_ref[m],
                         preferred_element_type=jnp.float32)
            f2 = jax.lax.dot_general(
                a2_ref[m], h, (((1,), (1,)), ((), ())),
                preferred_element_type=jnp.float32)
            f2m = jnp.max(f2)
            c1_scr[m] = jnp.where(
                cols < _N, jnp.exp(f2 - f2m), 0.0).astype(jnp.bfloat16)
            c2_scr[m] = jnp.where(
                cols < _N, jnp.exp(_LEAK * (f2 - f2m)), 0.0).astype(jnp.bfloat16)
            t = f1 + f2m
            mx = jnp.maximum(t, _LEAK * t)
            r1_scr[m] = jnp.exp(t - mx).astype(jnp.bfloat16)
            r2_scr[m] = jnp.exp(_LEAK * t - mx).astype(jnp.bfloat16)

    adj_blk = adj_ref[...]
    adjb = adj_blk.astype(jnp.bfloat16)
    row0 = i * _BN
    cond_blk = cond_scr[pl.ds(row0, _BN), :]

    for m in range(_NM):
        r1 = r1_scr[m, pl.ds(row0, _BN), :]
        r2 = r2_scr[m, pl.ds(row0, _BN), :]
        p = jnp.maximum(r1 * c1_scr[m], r2 * c2_scr[m]) * adjb
        hp = jnp.dot(p, h_scr[m], preferred_element_type=jnp.float32)
        s = hp[:, _OUTS:_OUTS + 1]
        gamma = cond_blk[:, m][:, None]
        beta = cond_blk[:, _NM + m][:, None]
        v = (gamma / s) * hp[:, :_OUTS] + beta
        out_ref[:, m * _OUTS:(m + 1) * _OUTS] = jnp.where(
            v > 0, v, jnp.exp(jnp.minimum(v, 0.0)) - 1.0)


def kernel(x, adj, W, a1, a2, Wc, bc):
    a1r = a1.reshape(_NM, _OUTS, 1)
    a2r = a2.reshape(_NM, 1, _OUTS)
    bcr = bc.reshape(1, 2 * _NM)

    grid = (_NP // _BN,)
    out = pl.pallas_call(
        _cat_kernel,
        grid=grid,
        in_specs=[
            pl.BlockSpec((_NP, _INS), lambda i: (0, 0)),
            pl.BlockSpec((_BN, _NP), lambda i: (i, 0)),
            pl.BlockSpec((_NM, _INS, _OUTS), lambda i: (0, 0, 0)),
            pl.BlockSpec((_NM, _OUTS, 1), lambda i: (0, 0, 0)),
            pl.BlockSpec((_NM, 1, _OUTS), lambda i: (0, 0, 0)),
            pl.BlockSpec((_INS, 2 * _NM), lambda i: (0, 0)),
            pl.BlockSpec((1, 2 * _NM), lambda i: (0, 0)),
        ],
        out_specs=pl.BlockSpec((_BN, _NM * _OUTS), lambda i: (i, 0)),
        out_shape=jax.ShapeDtypeStruct((_N, _NM * _OUTS), jnp.float32),
        scratch_shapes=[
            pltpu.VMEM((_NM, _NP, _HA), jnp.bfloat16),
            pltpu.VMEM((_NM, _NP, 1), jnp.bfloat16),
            pltpu.VMEM((_NM, _NP, 1), jnp.bfloat16),
            pltpu.VMEM((_NM, 1, _NP), jnp.bfloat16),
            pltpu.VMEM((_NM, 1, _NP), jnp.bfloat16),
            pltpu.VMEM((_NP, 2 * _NM), jnp.float32),
        ],
    )(x, adj, W, a1r, a2r, Wc, bcr)
    return out


# speedup vs baseline: 1.1184x; 1.1184x over previous
import jax
import jax.numpy as jnp
from jax.experimental import pallas as pl
from jax.experimental.pallas import tpu as pltpu

_N = 2708
_INS = 512
_OUTS = 64
_NM = 4
_LEAK = 0.2
_NP = 2816
_BN = 352
_HA = 128


def _cat_kernel(x_ref, adj_ref, W_ref, a1_ref, a2_ref, Wc_ref, bc_ref,
                out_ref, h_scr, r1_scr, r2_scr, c1_scr, c2_scr, cond_scr):
    i = pl.program_id(0)

    @pl.when(i == 0)
    def _prologue():
        x = x_ref[...]
        cond_scr[...] = (
            jnp.dot(x, Wc_ref[...], preferred_element_type=jnp.float32)
            + bc_ref[...]
        )
        xb = x.astype(jnp.bfloat16)
        rows = jax.lax.broadcasted_iota(jnp.int32, (_NP, 1), 0)
        cols = jax.lax.broadcasted_iota(jnp.int32, (1, _NP), 1)
        for m in range(_NM):
            h = jnp.dot(xb, W_ref[m].astype(jnp.bfloat16),
                        preferred_element_type=jnp.float32)
            h = jnp.where(rows < _N, h, 0.0)
            h_scr[m, :, :_OUTS] = h.astype(jnp.bfloat16)
            h_scr[m, :, _OUTS:] = jnp.ones((_NP, _HA - _OUTS), jnp.bfloat16)
            f1 = jnp.dot(h, a1_ref[m],
                         preferred_element_type=jnp.float32)
            f2 = jax.lax.dot_general(
                a2_ref[m], h, (((1,), (1,)), ((), ())),
                preferred_element_type=jnp.float32)
            f2m = jnp.max(f2)
            c1_scr[m] = jnp.where(
                cols < _N, jnp.exp(f2 - f2m), 0.0).astype(jnp.bfloat16)
            c2_scr[m] = jnp.where(
                cols < _N, jnp.exp(_LEAK * (f2 - f2m)), 0.0).astype(jnp.bfloat16)
            t = f1 + f2m
            mx = jnp.maximum(t, _LEAK * t)
            r1_scr[m] = jnp.exp(t - mx).astype(jnp.bfloat16)
            r2_scr[m] = jnp.exp(_LEAK * t - mx).astype(jnp.bfloat16)

    adj_blk = adj_ref[...]
    adjb = adj_blk.astype(jnp.bfloat16)
    row0 = i * _BN
    cond_blk = cond_scr[pl.ds(row0, _BN), :]

    for m in range(_NM):
        r1 = r1_scr[m, pl.ds(row0, _BN), :]
        r2 = r2_scr[m, pl.ds(row0, _BN), :]
        p = jnp.maximum(r1 * c1_scr[m], r2 * c2_scr[m]) * adjb
        hp = jnp.dot(p, h_scr[m], preferred_element_type=jnp.float32)
        s = hp[:, _OUTS:_OUTS + 1]
        gamma = cond_blk[:, m][:, None]
        beta = cond_blk[:, _NM + m][:, None]
        v = (gamma / s) * hp[:, :_OUTS] + beta
        out_ref[:, m * _OUTS:(m + 1) * _OUTS] = jnp.where(
            v > 0, v, jnp.exp(jnp.minimum(v, 0.0)) - 1.0)


def kernel(x, adj, W, a1, a2, Wc, bc):
    a1r = a1.reshape(_NM, _OUTS, 1)
    a2r = a2.reshape(_NM, 1, _OUTS)
    bcr = bc.reshape(1, 2 * _NM)

    grid = (_NP // _BN,)
    out = pl.pallas_call(
        _cat_kernel,
        grid=grid,
        in_specs=[
            pl.BlockSpec((_NP, _INS), lambda i: (0, 0)),
            pl.BlockSpec((_BN, _NP), lambda i: (i, 0)),
            pl.BlockSpec((_NM, _INS, _OUTS), lambda i: (0, 0, 0)),
            pl.BlockSpec((_NM, _OUTS, 1), lambda i: (0, 0, 0)),
            pl.BlockSpec((_NM, 1, _OUTS), lambda i: (0, 0, 0)),
            pl.BlockSpec((_INS, 2 * _NM), lambda i: (0, 0)),
            pl.BlockSpec((1, 2 * _NM), lambda i: (0, 0)),
        ],
        out_specs=pl.BlockSpec((_BN, _NM * _OUTS), lambda i: (i, 0)),
        out_shape=jax.ShapeDtypeStruct((_N, _NM * _OUTS), jnp.float32),
        scratch_shapes=[
            pltpu.VMEM((_NM, _NP, _HA), jnp.bfloat16),
            pltpu.VMEM((_NM, _NP, 1), jnp.bfloat16),
            pltpu.VMEM((_NM, _NP, 1), jnp.bfloat16),
            pltpu.VMEM((_NM, 1, _NP), jnp.bfloat16),
            pltpu.VMEM((_NM, 1, _NP), jnp.bfloat16),
            pltpu.VMEM((_NP, 2 * _NM), jnp.float32),
        ],
    )(x, adj, W, a1r, a2r, Wc, bcr)
    return out
